# baseline (speedup 1.0000x reference)
import functools
import math

import jax
import jax.numpy as jnp
from jax import lax
from jax.experimental import pallas as pl
from jax.experimental.pallas import tpu as pltpu

H = 16
K = 8


def _msla_body(x_ref, wq_ref, bq_ref, wv_ref, bv_ref, latc_ref, latp_ref,
               wo_ref, bo_ref, o_ref, logits_s, v_s, *, num_k, scale, num_h):
    hi = pl.program_id(2)
    dn = (((1,), (1,)), ((), ()))

    logits = logits_s[...]
    v_prev = v_s[...]

    work = logits
    mask = jnp.zeros(logits.shape, jnp.bool_)
    mx = None
    z = None
    for k in range(num_k):
        m = jnp.max(work, axis=1, keepdims=True)
        if k == 0:
            mx = m
            z = jnp.ones_like(m)
        else:
            z = z + jnp.exp(m - mx)
        sel = work == m
        mask = jnp.logical_or(mask, sel)
        work = jnp.where(sel, -jnp.inf, work)

    p = jnp.where(mask, jnp.exp(logits - mx), 0.0) / z

    weighted = lax.dot_general(p, latp_ref[0], (((1,), (0,)), ((), ())),
                               preferred_element_type=jnp.float32)
    head = weighted + v_prev
    contrib = lax.dot_general(head, wo_ref[...], dn,
                              preferred_element_type=jnp.float32)

    x = x_ref[0]
    q = lax.dot_general(x, wq_ref[...], dn,
                        preferred_element_type=jnp.float32) + bq_ref[0]
    v_new = lax.dot_general(x, wv_ref[...], dn,
                            preferred_element_type=jnp.float32) + bv_ref[0]
    logits_s[...] = lax.dot_general(q, latc_ref[0], dn,
                                    preferred_element_type=jnp.float32) * scale
    v_s[...] = v_new

    @pl.when(hi == 1)
    def _():
        o_ref[0] = contrib + bo_ref[...]

    @pl.when(hi > 1)
    def _():
        o_ref[0] += contrib


def kernel(hidden_states, Wq, bq, Wk, bk, Wv, bv, Wo, bo, latent_keys):
    del Wk, bk
    b, t, d = hidden_states.shape
    hd = d // H
    l = latent_keys.shape[0]
    tb = 512
    scale = 1.0 / math.sqrt(hd)

    bq_r = bq.reshape(H, 1, hd)
    bv_r = bv.reshape(H, 1, hd)
    lat_r = latent_keys.reshape(l, H, hd).transpose(1, 0, 2)
    bo_r = bo.reshape(1, d)

    def cur(hi):
        return jnp.minimum(hi, H - 1)

    def prev(hi):
        return jnp.maximum(hi - 1, 0)

    grid = (b, t // tb, H + 1)
    body = functools.partial(_msla_body, num_k=K, scale=scale, num_h=H)
    out = pl.pallas_call(
        body,
        grid=grid,
        in_specs=[
            pl.BlockSpec((1, tb, d), lambda bi, ti, hi: (bi, ti, 0)),
            pl.BlockSpec((hd, d), lambda bi, ti, hi: (cur(hi), 0)),
            pl.BlockSpec((1, 1, hd), lambda bi, ti, hi: (cur(hi), 0, 0)),
            pl.BlockSpec((hd, d), lambda bi, ti, hi: (cur(hi), 0)),
            pl.BlockSpec((1, 1, hd), lambda bi, ti, hi: (cur(hi), 0, 0)),
            pl.BlockSpec((1, l, hd), lambda bi, ti, hi: (cur(hi), 0, 0)),
            pl.BlockSpec((1, l, hd), lambda bi, ti, hi: (prev(hi), 0, 0)),
            pl.BlockSpec((d, hd), lambda bi, ti, hi: (0, prev(hi))),
            pl.BlockSpec((1, d), lambda bi, ti, hi: (0, 0)),
        ],
        out_specs=pl.BlockSpec((1, tb, d), lambda bi, ti, hi: (bi, ti, 0)),
        out_shape=jax.ShapeDtypeStruct((b, t, d), jnp.float32),
        scratch_shapes=[
            pltpu.MemorySpace.VMEM((tb, l), jnp.float32),
            pltpu.MemorySpace.VMEM((tb, hd), jnp.float32),
        ],
        compiler_params=pltpu.CompilerParams(
            dimension_semantics=("parallel", "parallel", "arbitrary"),
        ),
    )(hidden_states, Wq, bq_r, Wv, bv_r, lat_r, lat_r, Wo, bo_r)
    return out
